# baseline (speedup 1.0000x reference)
import functools

import jax
import jax.numpy as jnp
from jax import lax
from jax.experimental import pallas as pl
from jax.experimental.pallas import tpu as pltpu
from jax.experimental.pallas import tpu_sc as plsc

NC = 2
NS = 16
NW = NC * NS
L = 16
V = 1000000
D = 64
NCOLT = (V + 127) // 128
HCAP = 8192
SORTCAP = HCAP + 246 * L
NBUCKET = 256


def _sweep_body(nb, tab_t, tail_hbm, idx_hbm, rows_out, idx_v, hv, hs, sv, ss,
                bufA, bufB, tailbuf, stage, istage, counts, starts, cursor,
                sem, psemA, psemB):
    wid = lax.axis_index("s") * NC + lax.axis_index("c")
    iota = lax.iota(jnp.int32, L)
    lo_col = wid * NCOLT // NW
    hi_col = (wid + 1) * NCOLT // NW
    lo_v = lo_col * 128
    hi_v = hi_col * 128
    dummy_base = nb * D

    pltpu.sync_copy(idx_hbm, idx_v)

    def precount(j, acc):
        for k in range(8):
            v = idx_v[j, pl.ds(k * L, L)]
            m = (v >= lo_v) & (v < hi_v)
            acc = acc + m.astype(jnp.int32)
        return acc

    nh_tot = jnp.sum(lax.fori_loop(0, nb // 128, precount,
                                   jnp.zeros((L,), jnp.int32)))
    npass = jnp.where(nh_tot > HCAP, nb // HCAP, 1)
    csize = nb // npass

    def one_pass(p, gdone):
        base_row = p * (csize // 128)

        def filt(j, ptr):
            row = base_row + j
            for k in range(8):
                v = idx_v[row, pl.ds(k * L, L)]
                m = (v >= lo_v) & (v < hi_v)
                plsc.store_compressed(hv.at[pl.ds(ptr, L)], v, mask=m)
                plsc.store_compressed(
                    hs.at[pl.ds(ptr, L)], row * 128 + k * L + iota, mask=m)
                ptr = ptr + jnp.sum(m.astype(jnp.int32))
            return ptr

        nh = lax.fori_loop(0, csize // 128, filt, jnp.int32(0))

        def zero_b(c, _):
            counts[c] = 0
            return 0
        lax.fori_loop(0, NBUCKET, zero_b, 0)

        def count_step(i, _):
            v = hv[pl.ds(i * L, L)]
            cl = jnp.where(i * L + iota < nh, (v >> 7) - lo_col, NBUCKET - 1)
            for l in range(L):
                c = cl[l]
                counts[c] = counts[c] + 1
            return 0
        lax.fori_loop(0, (nh + L - 1) // L, count_step, 0)

        def cumsum_b(c, acc):
            cnt = counts[c]
            starts[c] = acc
            cursor[c] = acc
            return acc + ((cnt + L - 1) // L) * L
        lax.fori_loop(0, NBUCKET, cumsum_b, jnp.int32(0))

        lane0 = iota == 0

        def scatter_step(i, _):
            v = hv[pl.ds(i * L, L)]
            s = hs[pl.ds(i * L, L)]
            cl = jnp.where(i * L + iota < nh, (v >> 7) - lo_col, NBUCKET - 1)
            for l in range(L):
                c = cl[l]
                pos = cursor[c]
                cursor[c] = pos + 1
                posv = jnp.full((L,), pos, jnp.int32)
                plsc.store_scatter(sv, [posv],
                                   jnp.full((L,), v[l], jnp.int32), mask=lane0)
                plsc.store_scatter(ss, [posv],
                                   jnp.full((L,), s[l], jnp.int32), mask=lane0)
            return 0
        lax.fori_loop(0, (nh + L - 1) // L, scatter_step, 0)

        def start_col(c, buf, psem):
            cc = jnp.minimum(c, NCOLT - 2)
            pltpu.async_copy(
                tab_t.at[:, pl.ds(pl.multiple_of(cc * 128, 128), 128)],
                buf, psem)

        def wait_col(c, buf, psem):
            cc = jnp.minimum(c, NCOLT - 2)
            pltpu.make_async_copy(
                tab_t.at[:, pl.ds(pl.multiple_of(cc * 128, 128), 128)],
                buf, psem).wait()

        start_col(lo_col, bufA, psemA)

        def per_col(buf, col, gdone):
            start_v = col * 128
            cl = col - lo_col
            b0 = starts[cl]
            rcnt = counts[cl]

            def group(g, gdone):
                sp = lax.rem(gdone, 8)
                h = lax.rem(gdone // 8, 2)

                def drain_entry(_):
                    pltpu.make_async_copy(
                        rows_out.at[pl.ds(0, 128), :],
                        stage.at[pl.ds(0, 128), :], sem).wait()
                    return 0
                lax.cond((sp == 0) & (gdone // 8 >= 2), drain_entry,
                         lambda _: 0, 0)

                gb = b0 + g * L
                sv16 = sv[pl.ds(gb, L)]
                ss16 = ss[pl.ds(gb, L)]
                m = (g * L + iota) < rcnt
                vloc = jnp.clip(sv16 - start_v, 0, 127)
                istail = sv16 >= vfull
                tloc = jnp.clip(sv16 - vfull, 0, (V - vfull) - 1)
                rowv = h * 128 + sp * L + iota
                for d in range(D):
                    x = plsc.load_gather(
                        buf, [jnp.full((L,), d, jnp.int32), vloc])
                    tflat = tloc * D + d
                    xt = plsc.load_gather(
                        tailbuf, [tflat >> 7, tflat & 127])
                    x = jnp.where(istail, xt, x)
                    plsc.store_scatter(
                        stage, [rowv, jnp.full((L,), d, jnp.int32)], x)
                istage[h, pl.ds(sp * L, L)] = jnp.where(m, ss16, nb)

                def flush(_):
                    pltpu.async_copy(stage.at[pl.ds(h * 128, 128), :],
                                     rows_out.at[istage.at[h]], sem)
                    return 0
                lax.cond(sp == 7, flush, lambda _: 0, 0)
                return gdone + 1

            return lax.fori_loop(0, (rcnt + L - 1) // L, group, gdone)

        def two_cols(i, gdone):
            cA = lo_col + 2 * i
            cB = jnp.minimum(cA + 1, hi_col - 1)
            cN = jnp.minimum(cA + 2, hi_col - 1)
            wait_col(cA, bufA, psemA)
            start_col(cB, bufB, psemB)
            gdone = per_col(bufA, cA, gdone)
            wait_col(cB, bufB, psemB)
            start_col(cN, bufA, psemA)
            return per_col(bufB, cB, gdone)

        ncols2 = (hi_col - lo_col + 1) // 2
        gdone = lax.fori_loop(0, ncols2, two_cols, gdone)
        wait_col(hi_col - 1, bufA, psemA)
        return gdone

    gdone = lax.fori_loop(0, npass, one_pass, jnp.int32(0))

    def pflush(_):
        h = lax.rem(gdone // 8, 2)
        pltpu.async_copy(stage.at[pl.ds(h * 128, 128), :],
                         rows_out.at[istage.at[h]], sem)
        return 0
    lax.cond(lax.rem(gdone, 8) != 0, pflush, lambda _: 0, 0)
    fissued = gdone // 8 + (lax.rem(gdone, 8) != 0).astype(jnp.int32)
    fdrained = jnp.maximum(0, (gdone - 1) // 8 - 1) * (gdone >= 1).astype(jnp.int32)

    def fdrain(i, _):
        pltpu.make_async_copy(rows_out.at[pl.ds(0, 128), :],
                              stage.at[pl.ds(0, 128), :], sem).wait()
        return 0
    lax.fori_loop(0, fissued - fdrained, fdrain, 0)


def _dot_body(b_per_w, vin_g, w_g, bias_hbm, idxo_hbm, out_hbm,
              vin_v, w_v, bias_v, idxo_v, out_v, t0_v, t1_v, sem):
    wid = lax.axis_index("s") * NC + lax.axis_index("c")
    iota = lax.iota(jnp.int32, L)
    nb2 = 2 * b_per_w

    pltpu.sync_copy(vin_g.at[pl.ds(wid * b_per_w * D, b_per_w * D)], vin_v)
    pltpu.sync_copy(w_g.at[pl.ds(wid * nb2 * D, nb2 * D)], w_v)
    pltpu.sync_copy(idxo_hbm.at[wid], idxo_v)
    copies = []
    for j in range(nb2 // 128):
        copies.append(pltpu.async_copy(
            bias_hbm.at[idxo_v.at[j]], bias_v.at[pl.ds(j * 128, 128)], sem))
    for c in copies:
        c.wait()

    def group(g, carry):
        base = g * L
        for r in range(L):
            b = base + r
            p0 = None
            p1 = None
            for c in range(4):
                vin_c = vin_v[pl.ds(b * D + c * L, L)]
                m0 = vin_c * w_v[pl.ds(2 * b * D + c * L, L)]
                m1 = vin_c * w_v[pl.ds((2 * b + 1) * D + c * L, L)]
                p0 = m0 if p0 is None else p0 + m0
                p1 = m1 if p1 is None else p1 + m1
            t0_v[pl.ds(r * L, L)] = p0
            t1_v[pl.ds(r * L, L)] = p1
        row_base = iota * L
        dot0 = None
        dot1 = None
        for c in range(L):
            g0 = plsc.load_gather(t0_v, [row_base + c])
            g1 = plsc.load_gather(t1_v, [row_base + c])
            dot0 = g0 if dot0 is None else dot0 + g0
            dot1 = g1 if dot1 is None else dot1 + g1
        pos0 = 2 * (base + iota)
        pos1 = pos0 + 1
        t0 = dot0 + plsc.load_gather(bias_v, [pos0])
        t1 = dot1 + plsc.load_gather(bias_v, [pos1])
        s0 = 1.0 / (1.0 + jnp.exp(-t0))
        s1 = 1.0 / (1.0 + jnp.exp(-t1))
        plsc.store_scatter(out_v, [pos0], s0)
        plsc.store_scatter(out_v, [pos1], s1)
        return carry

    lax.fori_loop(0, b_per_w // L, group, 0)
    pltpu.sync_copy(out_v, out_hbm.at[pl.ds(wid * nb2, nb2)])


def _make_sweep(nb):
    mesh = plsc.VectorSubcoreMesh(core_axis_name="c", subcore_axis_name="s")
    return pl.kernel(
        functools.partial(_sweep_body, nb),
        out_type=jax.ShapeDtypeStruct((nb + 1, 128), jnp.float32),
        mesh=mesh,
        compiler_params=pltpu.CompilerParams(
            needs_layout_passes=False, use_tc_tiling_on_sc=True),
        scratch_types=[
            pltpu.VMEM((nb // 128, 128), jnp.int32),
            pltpu.VMEM((HCAP + L,), jnp.int32),
            pltpu.VMEM((HCAP + L,), jnp.int32),
            pltpu.VMEM((SORTCAP,), jnp.int32),
            pltpu.VMEM((SORTCAP,), jnp.int32),
            pltpu.VMEM((D, 128), jnp.float32),
            pltpu.VMEM((D, 128), jnp.float32),
            pltpu.VMEM(((V - (V // 128) * 128) * D // 128, 128),
                       jnp.float32),
            pltpu.VMEM((256, 128), jnp.float32),
            pltpu.VMEM((2, 128), jnp.int32),
            pltpu.SMEM((NBUCKET,), jnp.int32),
            pltpu.SMEM((NBUCKET,), jnp.int32),
            pltpu.SMEM((NBUCKET,), jnp.int32),
            pltpu.SemaphoreType.DMA,
            pltpu.SemaphoreType.DMA,
            pltpu.SemaphoreType.DMA,
        ],
    )


def _make_dot(batch):
    b_per_w = batch // NW
    mesh = plsc.VectorSubcoreMesh(core_axis_name="c", subcore_axis_name="s")
    return pl.kernel(
        functools.partial(_dot_body, b_per_w),
        out_type=jax.ShapeDtypeStruct((batch * 2,), jnp.float32),
        mesh=mesh,
        compiler_params=pltpu.CompilerParams(needs_layout_passes=False),
        scratch_types=[
            pltpu.VMEM((b_per_w * D,), jnp.float32),
            pltpu.VMEM((2 * b_per_w * D,), jnp.float32),
            pltpu.VMEM((2 * b_per_w,), jnp.float32),
            pltpu.VMEM((2 * b_per_w // 128, 128), jnp.int32),
            pltpu.VMEM((2 * b_per_w,), jnp.float32),
            pltpu.VMEM((L * L,), jnp.float32),
            pltpu.VMEM((L * L,), jnp.float32),
            pltpu.SemaphoreType.DMA,
        ],
    )


def kernel(x, emb_in, emb_out_w, emb_out_b):
    batch = x.shape[0]
    assert emb_in.shape == (V, D) and batch % (NW * L) == 0

    idx_in = x[:, 0].reshape(batch // 128, 128)
    idx_out = x[:, 1:3].reshape(2 * batch // 128, 128)
    idxo3 = idx_out.reshape(NW, 2 * batch // NW // 128, 128)
    bias_lin = emb_out_b.reshape(V)
    vfull = (V // 128) * 128
    tail_in = emb_in[vfull:].reshape((V - vfull) * D // 128, 128)
    tail_w = emb_out_w[vfull:].reshape((V - vfull) * D // 128, 128)

    vin_g = _make_sweep(batch)(emb_in.T, tail_in, idx_in)
    w_g = _make_sweep(2 * batch)(emb_out_w.T, tail_w, idx_out)
    vin_lin = vin_g[:, :D].reshape(-1)
    w_lin = w_g[:, :D].reshape(-1)
    out = _make_dot(batch)(vin_lin, w_lin, bias_lin, idxo3)
    return out.reshape(batch, 2)


# speedup vs baseline: 2.1542x; 2.1542x over previous
import functools

import jax
import jax.numpy as jnp
from jax import lax
from jax.experimental import pallas as pl
from jax.experimental.pallas import tpu as pltpu
from jax.experimental.pallas import tpu_sc as plsc

NC = 2
NS = 16
NW = NC * NS
L = 16
V = 1000000
D = 64
NCOLT = (V + 127) // 128
VFULL = (V // 128) * 128
IDX_CHUNK = 128


def _trans_body(tab_t, tail_hbm, out2d, bufA, bufB, tailbuf, tbuf,
                psemA, psemB, wsem):
    wid = lax.axis_index("s") * NC + lax.axis_index("c")
    iota = lax.iota(jnp.int32, L)
    lo_col = wid * NCOLT // NW
    hi_col = (wid + 1) * NCOLT // NW

    pltpu.sync_copy(tail_hbm, tailbuf)

    def start_col(c, buf, psem):
        cc = jnp.minimum(c, NCOLT - 2)
        pltpu.async_copy(
            tab_t.at[:, pl.ds(pl.multiple_of(cc * 128, 128), 128)],
            buf, psem)

    def wait_col(c, buf, psem):
        cc = jnp.minimum(c, NCOLT - 2)
        pltpu.make_async_copy(
            tab_t.at[:, pl.ds(pl.multiple_of(cc * 128, 128), 128)],
            buf, psem).wait()

    start_col(lo_col, bufA, psemA)

    def block(buf, col, cnt):
        def drain(_):
            pltpu.make_async_copy(out2d.at[pl.ds(0, 64), :],
                                  tbuf.at[pl.ds(0, 64), :], wsem).wait()
            return 0
        lax.cond(cnt >= 2, drain, lambda _: 0, 0)

        h = lax.rem(cnt, 2)
        base = h * 64
        istail = col >= NCOLT - 1
        for v in range(128):
            for dc in range(4):
                x = plsc.load_gather(
                    buf, [dc * L + iota, jnp.full((L,), v, jnp.int32)])
                if v < V - VFULL:
                    flat = v * D + dc * L
                    xt = plsc.load_gather(
                        tailbuf, [jnp.full((L,), flat // 128, jnp.int32),
                                  flat % 128 + iota])
                    x = jnp.where(istail, xt, x)
                tbuf[base + v // 2, pl.ds((v % 2) * D + dc * L, L)] = x
        pltpu.async_copy(tbuf.at[pl.ds(base, 64), :],
                         out2d.at[pl.ds(col * 64, 64), :], wsem)
        return cnt + 1

    def two_cols(i, cnt):
        cA = lo_col + 2 * i
        cB = jnp.minimum(cA + 1, hi_col - 1)
        cN = jnp.minimum(cA + 2, hi_col - 1)
        wait_col(cA, bufA, psemA)
        start_col(cB, bufB, psemB)
        cnt = block(bufA, cA, cnt)
        wait_col(cB, bufB, psemB)
        start_col(cN, bufA, psemA)
        return block(bufB, cB, cnt)

    ncols2 = (hi_col - lo_col + 1) // 2
    cnt = lax.fori_loop(0, ncols2, two_cols, jnp.int32(0))
    wait_col(hi_col - 1, bufA, psemA)

    def fdrain(i, _):
        pltpu.make_async_copy(out2d.at[pl.ds(0, 64), :],
                              tbuf.at[pl.ds(0, 64), :], wsem).wait()
        return 0
    lax.fori_loop(0, jnp.minimum(cnt, 2), fdrain, 0)


def _dot_body(b_per_w, idx_in_hbm, idx_out_hbm, emb_in_hbm, emb_w_hbm,
              bias_hbm, out_hbm, idxin_v, idxout_v, vin_v, w_v, bias_v,
              out_v, t0_v, t1_v, sem):
    wid = lax.axis_index("s") * NC + lax.axis_index("c")
    n_in_chunks = b_per_w // IDX_CHUNK
    n_out_chunks = 2 * b_per_w // IDX_CHUNK

    pltpu.sync_copy(idx_in_hbm.at[wid], idxin_v)
    pltpu.sync_copy(idx_out_hbm.at[wid], idxout_v)

    copies = []
    for j in range(n_in_chunks):
        copies.append(pltpu.async_copy(
            emb_in_hbm.at[idxin_v.at[j]],
            vin_v.at[pl.ds(j * IDX_CHUNK, IDX_CHUNK)], sem))
    for j in range(n_out_chunks):
        copies.append(pltpu.async_copy(
            emb_w_hbm.at[idxout_v.at[j]],
            w_v.at[pl.ds(j * IDX_CHUNK, IDX_CHUNK)], sem))
    for j in range(n_out_chunks):
        copies.append(pltpu.async_copy(
            bias_hbm.at[idxout_v.at[j]],
            bias_v.at[pl.ds(j * IDX_CHUNK, IDX_CHUNK)], sem))
    for c in copies:
        c.wait()

    iota = lax.iota(jnp.int32, L)

    def group(g, carry):
        base = g * L
        for r in range(L):
            b = base + r
            p0 = None
            p1 = None
            for c in range(4):
                sl = pl.ds(c * L, L)
                vin_c = vin_v[b, sl]
                m0 = vin_c * w_v[2 * b, sl]
                m1 = vin_c * w_v[2 * b + 1, sl]
                p0 = m0 if p0 is None else p0 + m0
                p1 = m1 if p1 is None else p1 + m1
            t0_v[pl.ds(r * L, L)] = p0
            t1_v[pl.ds(r * L, L)] = p1
        row_base = iota * L
        dot0 = None
        dot1 = None
        for c in range(L):
            g0 = plsc.load_gather(t0_v, [row_base + c])
            g1 = plsc.load_gather(t1_v, [row_base + c])
            dot0 = g0 if dot0 is None else dot0 + g0
            dot1 = g1 if dot1 is None else dot1 + g1
        pos0 = 2 * (base + iota)
        pos1 = pos0 + 1
        t0 = dot0 + plsc.load_gather(bias_v, [pos0])
        t1 = dot1 + plsc.load_gather(bias_v, [pos1])
        s0 = 1.0 / (1.0 + jnp.exp(-t0))
        s1 = 1.0 / (1.0 + jnp.exp(-t1))
        plsc.store_scatter(out_v, [pos0], s0)
        plsc.store_scatter(out_v, [pos1], s1)
        return carry

    lax.fori_loop(0, b_per_w // L, group, 0)
    pltpu.sync_copy(out_v, out_hbm.at[pl.ds(wid * 2 * b_per_w, 2 * b_per_w)])


def _make_trans():
    mesh = plsc.VectorSubcoreMesh(core_axis_name="c", subcore_axis_name="s")
    return pl.kernel(
        _trans_body,
        out_type=jax.ShapeDtypeStruct((NCOLT * 64, 128), jnp.float32),
        mesh=mesh,
        compiler_params=pltpu.CompilerParams(
            needs_layout_passes=False, use_tc_tiling_on_sc=True),
        scratch_types=[
            pltpu.VMEM((D, 128), jnp.float32),
            pltpu.VMEM((D, 128), jnp.float32),
            pltpu.VMEM(((V - VFULL) * D // 128, 128), jnp.float32),
            pltpu.VMEM((128, 128), jnp.float32),
            pltpu.SemaphoreType.DMA,
            pltpu.SemaphoreType.DMA,
            pltpu.SemaphoreType.DMA,
        ],
    )


def _make_dot(batch):
    b_per_w = batch // NW
    mesh = plsc.VectorSubcoreMesh(core_axis_name="c", subcore_axis_name="s")
    return pl.kernel(
        functools.partial(_dot_body, b_per_w),
        out_type=jax.ShapeDtypeStruct((batch * 2,), jnp.float32),
        mesh=mesh,
        compiler_params=pltpu.CompilerParams(
            needs_layout_passes=False, use_tc_tiling_on_sc=False),
        scratch_types=[
            pltpu.VMEM((b_per_w // IDX_CHUNK, IDX_CHUNK), jnp.int32),
            pltpu.VMEM((2 * b_per_w // IDX_CHUNK, IDX_CHUNK), jnp.int32),
            pltpu.VMEM((b_per_w, D), jnp.float32),
            pltpu.VMEM((2 * b_per_w, D), jnp.float32),
            pltpu.VMEM((2 * b_per_w,), jnp.float32),
            pltpu.VMEM((2 * b_per_w,), jnp.float32),
            pltpu.VMEM((L * L,), jnp.float32),
            pltpu.VMEM((L * L,), jnp.float32),
            pltpu.SemaphoreType.DMA,
        ],
    )


def kernel(x, emb_in, emb_out_w, emb_out_b):
    batch = x.shape[0]
    assert emb_in.shape == (V, D) and batch % (NW * L) == 0
    b_per_w = batch // NW

    idx_in = x[:, 0].reshape(NW, b_per_w // IDX_CHUNK, IDX_CHUNK)
    idx_out = x[:, 1:3].reshape(NW, 2 * b_per_w // IDX_CHUNK, IDX_CHUNK)
    bias_lin = emb_out_b.reshape(V)
    tail_in = emb_in[VFULL:].reshape((V - VFULL) * D // 128, 128)
    tail_w = emb_out_w[VFULL:].reshape((V - VFULL) * D // 128, 128)

    trans = _make_trans()
    t_in = trans(emb_in.T, tail_in).reshape(NCOLT * 128, D)
    t_w = trans(emb_out_w.T, tail_w).reshape(NCOLT * 128, D)

    out_flat = _make_dot(batch)(idx_in, idx_out, t_in, t_w, bias_lin)
    return out_flat.reshape(batch, 2)


# speedup vs baseline: 7.0339x; 3.2653x over previous
import functools

import jax
import jax.numpy as jnp
from jax import lax
from jax.experimental import pallas as pl
from jax.experimental.pallas import tpu as pltpu
from jax.experimental.pallas import tpu_sc as plsc

NC = 2
NS = 16
NW = NC * NS
LANES = 16
IDX_CHUNK = 128


def _skipgram_body(b_per_w, idx_in_hbm, idx_out_hbm, emb_in_hbm, emb_w_hbm,
                   bias_hbm, out_hbm, idxin_v, idxout_v, vin_v, w_v, bias_v,
                   out_v, t0_v, t1_v, sem):
    wid = lax.axis_index("s") * NC + lax.axis_index("c")
    n_in_chunks = b_per_w // IDX_CHUNK
    n_out_chunks = 2 * b_per_w // IDX_CHUNK

    pltpu.sync_copy(idx_in_hbm.at[wid], idxin_v)
    pltpu.sync_copy(idx_out_hbm.at[wid], idxout_v)

    copies = []
    for j in range(n_in_chunks):
        copies.append(pltpu.async_copy(
            emb_in_hbm.at[idxin_v.at[j]],
            vin_v.at[pl.ds(j * IDX_CHUNK, IDX_CHUNK)], sem))
    for j in range(n_out_chunks):
        copies.append(pltpu.async_copy(
            emb_w_hbm.at[idxout_v.at[j]],
            w_v.at[pl.ds(j * IDX_CHUNK, IDX_CHUNK)], sem))
    for j in range(n_out_chunks):
        copies.append(pltpu.async_copy(
            bias_hbm.at[idxout_v.at[j]],
            bias_v.at[pl.ds(j * IDX_CHUNK, IDX_CHUNK)], sem))
    for c in copies:
        c.wait()

    iota = lax.iota(jnp.int32, LANES)

    def group(g, carry):
        base = g * LANES
        for r in range(LANES):
            b = base + r
            p0 = None
            p1 = None
            for c in range(4):
                sl = pl.ds(c * LANES, LANES)
                vin_c = vin_v[b, sl]
                m0 = vin_c * w_v[2 * b, sl]
                m1 = vin_c * w_v[2 * b + 1, sl]
                p0 = m0 if p0 is None else p0 + m0
                p1 = m1 if p1 is None else p1 + m1
            t0_v[pl.ds(r * LANES, LANES)] = p0
            t1_v[pl.ds(r * LANES, LANES)] = p1
        row_base = iota * LANES
        dot0 = None
        dot1 = None
        for c in range(LANES):
            g0 = plsc.load_gather(t0_v, [row_base + c])
            g1 = plsc.load_gather(t1_v, [row_base + c])
            dot0 = g0 if dot0 is None else dot0 + g0
            dot1 = g1 if dot1 is None else dot1 + g1
        pos0 = 2 * (base + iota)
        pos1 = pos0 + 1
        t0 = dot0 + plsc.load_gather(bias_v, [pos0])
        t1 = dot1 + plsc.load_gather(bias_v, [pos1])
        s0 = 1.0 / (1.0 + jnp.exp(-t0))
        s1 = 1.0 / (1.0 + jnp.exp(-t1))
        plsc.store_scatter(out_v, [pos0], s0)
        plsc.store_scatter(out_v, [pos1], s1)
        return carry

    lax.fori_loop(0, b_per_w // LANES, group, 0)

    pltpu.sync_copy(out_v, out_hbm.at[pl.ds(wid * 2 * b_per_w, 2 * b_per_w)])


def kernel(x, emb_in, emb_out_w, emb_out_b):
    batch = x.shape[0]
    vocab, embed = emb_in.shape
    assert batch % (NW * LANES) == 0 and embed == 4 * LANES
    b_per_w = batch // NW

    idx_in = x[:, 0].reshape(NW, b_per_w // IDX_CHUNK, IDX_CHUNK)
    idx_out = x[:, 1:3].reshape(NW, 2 * b_per_w // IDX_CHUNK, IDX_CHUNK)
    bias_flat = emb_out_b.reshape(vocab)

    mesh = plsc.VectorSubcoreMesh(core_axis_name="c", subcore_axis_name="s")
    run = pl.kernel(
        functools.partial(_skipgram_body, b_per_w),
        out_type=jax.ShapeDtypeStruct((batch * 2,), jnp.float32),
        mesh=mesh,
        compiler_params=pltpu.CompilerParams(
            needs_layout_passes=False, use_tc_tiling_on_sc=False),
        scratch_types=[
            pltpu.VMEM((b_per_w // IDX_CHUNK, IDX_CHUNK), jnp.int32),
            pltpu.VMEM((2 * b_per_w // IDX_CHUNK, IDX_CHUNK), jnp.int32),
            pltpu.VMEM((b_per_w, embed), jnp.float32),
            pltpu.VMEM((2 * b_per_w, embed), jnp.float32),
            pltpu.VMEM((2 * b_per_w,), jnp.float32),
            pltpu.VMEM((2 * b_per_w,), jnp.float32),
            pltpu.VMEM((LANES * LANES,), jnp.float32),
            pltpu.VMEM((LANES * LANES,), jnp.float32),
            pltpu.SemaphoreType.DMA,
        ],
    )
    out_flat = run(idx_in, idx_out, emb_in, emb_out_w, bias_flat)
    return out_flat.reshape(batch, 2)
